# baseline (speedup 1.0000x reference)
import jax
import jax.numpy as jnp
from jax.experimental import pallas as pl
from jax.experimental.pallas import tpu as pltpu

_D_EMB = 64


def _pos_block_kernel(e_ref, w_ref, b_ref, o_ref):
    o_ref[0] = (
        jnp.dot(e_ref[...], w_ref[...], preferred_element_type=jnp.float32)
        + b_ref[...]
    )


def kernel(x, embedding, W, b):
    B, S, D = x.shape
    bs = 512
    ns = S // bs
    b2 = b.reshape(1, D)
    return pl.pallas_call(
        _pos_block_kernel,
        grid=(ns, B),
        in_specs=[
            pl.BlockSpec((bs, _D_EMB), lambda i, j: (i, 0)),
            pl.BlockSpec((_D_EMB, D), lambda i, j: (0, 0)),
            pl.BlockSpec((1, D), lambda i, j: (0, 0)),
        ],
        out_specs=pl.BlockSpec((1, bs, D), lambda i, j: (j, i, 0)),
        out_shape=jax.ShapeDtypeStruct((B, S, D), jnp.float32),
        compiler_params=pltpu.CompilerParams(
            dimension_semantics=("parallel", "parallel"),
        ),
    )(embedding, W, b2)


# speedup vs baseline: 1.2750x; 1.2750x over previous
import jax
import jax.numpy as jnp
from jax.experimental import pallas as pl
from jax.experimental.pallas import tpu as pltpu

_D_EMB = 64


def _pos_block_kernel(e_ref, w_ref, b_ref, o_ref):
    o_ref[0] = (
        jnp.dot(e_ref[...], w_ref[...], preferred_element_type=jnp.float32)
        + b_ref[...]
    )


def kernel(x, embedding, W, b):
    B, S, D = x.shape
    bs = 1024
    ns = S // bs
    b2 = b.reshape(1, D)
    return pl.pallas_call(
        _pos_block_kernel,
        grid=(ns, B),
        in_specs=[
            pl.BlockSpec((bs, _D_EMB), lambda i, j: (i, 0)),
            pl.BlockSpec((_D_EMB, D), lambda i, j: (0, 0)),
            pl.BlockSpec((1, D), lambda i, j: (0, 0)),
        ],
        out_specs=pl.BlockSpec((1, bs, D), lambda i, j: (j, i, 0)),
        out_shape=jax.ShapeDtypeStruct((B, S, D), jnp.float32),
        compiler_params=pltpu.CompilerParams(
            dimension_semantics=("parallel", "parallel"),
        ),
    )(embedding, W, b2)


# speedup vs baseline: 1.4695x; 1.1525x over previous
import jax
import jax.numpy as jnp
from jax.experimental import pallas as pl
from jax.experimental.pallas import tpu as pltpu

_D_EMB = 64


def _pos_block_kernel(e_ref, w_ref, b_ref, o_ref):
    o_ref[0] = (
        jnp.dot(e_ref[...], w_ref[...], preferred_element_type=jnp.float32)
        + b_ref[...]
    )


def kernel(x, embedding, W, b):
    B, S, D = x.shape
    bs = 2048
    ns = S // bs
    b2 = b.reshape(1, D)
    return pl.pallas_call(
        _pos_block_kernel,
        grid=(ns, B),
        in_specs=[
            pl.BlockSpec((bs, _D_EMB), lambda i, j: (i, 0)),
            pl.BlockSpec((_D_EMB, D), lambda i, j: (0, 0)),
            pl.BlockSpec((1, D), lambda i, j: (0, 0)),
        ],
        out_specs=pl.BlockSpec((1, bs, D), lambda i, j: (j, i, 0)),
        out_shape=jax.ShapeDtypeStruct((B, S, D), jnp.float32),
        compiler_params=pltpu.CompilerParams(
            dimension_semantics=("parallel", "parallel"),
        ),
    )(embedding, W, b2)
